# Initial kernel scaffold; baseline (speedup 1.0000x reference)
import jax
import jax.numpy as jnp
from jax.experimental import pallas as pl


def kernel(user_emb, item_emb, edge_index, edge_weight):
    raise NotImplementedError("write your pallas kernel here")


# speedup vs baseline: 4.0647x; 4.0647x over previous
import functools

import jax
import jax.numpy as jnp
from jax import lax
from jax.experimental import pallas as pl
from jax.experimental.pallas import tpu as pltpu
from jax.experimental.pallas import tpu_sc as plsc

NC = 2
NS = 16
LANES = 16
DH = 32
RB = 128
SB = 8
N_LAYERS = 3
F32 = jnp.float32
I32 = jnp.int32


@functools.lru_cache(maxsize=None)
def _build(Npad, NBp):
  NSB = NBp // SB
  RPT = Npad // NS
  NRC = RPT // RB

  mesh = plsc.VectorSubcoreMesh(
      core_axis_name="c", subcore_axis_name="s", num_cores=NC, num_subcores=NS
  )

  @functools.partial(
      pl.kernel,
      out_type=(
          jax.ShapeDtypeStruct((NC, Npad, DH), F32),
          jax.ShapeDtypeStruct((NC, Npad, DH), F32),
          jax.ShapeDtypeStruct((NC, Npad, DH), F32),
      ),
      mesh=mesh,
      compiler_params=pltpu.CompilerParams(needs_layout_passes=False, use_tc_tiling_on_sc=False),
      scratch_types=dict(
          acc=pltpu.VMEM_SHARED((Npad, DH), F32),
          srcb=pltpu.VMEM((SB, RB), I32),
          dstb=pltpu.VMEM((SB, RB), I32),
          wb=pltpu.VMEM((SB * RB,), F32),
          rows=pltpu.VMEM((RB, DH), F32),
          accb=pltpu.VMEM((RB, DH), F32),
          sumb=pltpu.VMEM((RB, DH), F32),
          zb=pltpu.VMEM((RB, DH), F32),
      ),
  )
  def k(x0, src3, dst3, w3, xsum, xa, xb, acc, srcb, dstb, wb, rows, accb,
        sumb, zb):
    c = lax.axis_index("c")
    s = lax.axis_index("s")
    row0_t = s * RPT

    zeros = jnp.zeros((LANES,), F32)

    def zb_body(r, carry):
      zb[r, pl.ds(0, LANES)] = zeros
      zb[r, pl.ds(LANES, LANES)] = zeros
      return carry

    lax.fori_loop(0, RB, zb_body, 0)

    def zacc_body(kk, carry):
      pltpu.sync_copy(zb, acc.at[pl.ds(row0_t + kk * RB, RB), :])
      return carry

    lax.fori_loop(0, NRC, zacc_body, 0)
    plsc.subcore_barrier()

    def edge_pass(xprev):
      def sb_body(q, carry):
        base = q * SB
        pltpu.sync_copy(src3.at[s, pl.ds(base, SB), :], srcb)
        pltpu.sync_copy(dst3.at[s, pl.ds(base, SB), :], dstb)
        pltpu.sync_copy(w3.at[s, pl.ds(base * RB, SB * RB)], wb)

        def b_body(b, carry2):
          pltpu.sync_copy(xprev.at[c].at[srcb.at[b]], rows)
          woff = jnp.full((LANES,), b * RB, I32)

          def e_body(e, carry3):
            w = plsc.load_gather(wb, [woff + e])
            r0 = rows[e, pl.ds(0, LANES)]
            rows[e, pl.ds(0, LANES)] = r0 * w
            r1 = rows[e, pl.ds(LANES, LANES)]
            rows[e, pl.ds(LANES, LANES)] = r1 * w
            return carry3

          lax.fori_loop(0, RB, e_body, 0)
          pltpu.sync_copy(rows, acc.at[dstb.at[b]], add=True)
          return carry2

        lax.fori_loop(0, SB, b_body, 0)
        return carry

      lax.fori_loop(0, NSB, sb_body, 0)

    def writeback(layer, xnew):
      final = layer == N_LAYERS - 1
      scale = jnp.float32(1.0 / (N_LAYERS + 1))

      def k_body(kk, carry):
        row0 = row0_t + kk * RB
        pltpu.sync_copy(acc.at[pl.ds(row0, RB), :], accb)
        pltpu.sync_copy(zb, acc.at[pl.ds(row0, RB), :])
        if not final:
          pltpu.sync_copy(accb, xnew.at[c, pl.ds(row0, RB), :])
        if layer == 0:
          pltpu.sync_copy(x0.at[c, pl.ds(row0, RB), :], sumb)
        else:
          pltpu.sync_copy(xsum.at[c, pl.ds(row0, RB), :], sumb)

        def r_body(r, carry2):
          t0 = sumb[r, pl.ds(0, LANES)] + accb[r, pl.ds(0, LANES)]
          t1 = sumb[r, pl.ds(LANES, LANES)] + accb[r, pl.ds(LANES, LANES)]
          if final:
            t0 = t0 * scale
            t1 = t1 * scale
          sumb[r, pl.ds(0, LANES)] = t0
          sumb[r, pl.ds(LANES, LANES)] = t1
          return carry2

        lax.fori_loop(0, RB, r_body, 0)
        pltpu.sync_copy(sumb, xsum.at[c, pl.ds(row0, RB), :])
        return carry

      lax.fori_loop(0, NRC, k_body, 0)

    xprevs = [x0, xa, xb]
    xnews = [xa, xb, xa]
    for layer in range(N_LAYERS):
      edge_pass(xprevs[layer])
      plsc.subcore_barrier()
      writeback(layer, xnews[layer])
      plsc.subcore_barrier()

  return k


@jax.jit
def _lightgcn(user_emb, item_emb, edge_index, edge_weight):
  nu = user_emb.shape[0]
  ni = item_emb.shape[0]
  n = nu + ni
  ego = jnp.concatenate([user_emb, item_emb], axis=0)
  npad = -(-n // (NS * RB)) * (NS * RB)
  ego = jnp.pad(ego, ((0, npad - n), (0, 0)))
  x0 = jnp.stack([ego[:, :DH], ego[:, DH:]], axis=0)

  e = edge_index.shape[1]
  nbp = -(-e // (NS * RB * SB)) * SB
  epad = nbp * NS * RB
  src = jnp.pad(edge_index[0], (0, epad - e)).reshape(NS, nbp, RB)
  dst = jnp.pad(edge_index[1], (0, epad - e)).reshape(NS, nbp, RB)
  w = jnp.pad(edge_weight, (0, epad - e)).reshape(NS, nbp * RB)

  xsum, _, _ = _build(npad, nbp)(x0, src, dst, w)
  mean = jnp.concatenate([xsum[0, :n], xsum[1, :n]], axis=1)
  return mean[:nu], mean[nu:]


def kernel(user_emb, item_emb, edge_index, edge_weight):
  return _lightgcn(user_emb, item_emb, edge_index, edge_weight)


# speedup vs baseline: 5.5858x; 1.3742x over previous
import functools

import jax
import jax.numpy as jnp
from jax import lax
from jax.experimental import pallas as pl
from jax.experimental.pallas import tpu as pltpu
from jax.experimental.pallas import tpu_sc as plsc

NC = 2
NS = 16
LANES = 16
DH = 32
RB = 128
SB = 8
N_LAYERS = 3
F32 = jnp.float32
I32 = jnp.int32


@functools.lru_cache(maxsize=None)
def _build(Npad, NBp):
  NSB = NBp // SB
  RPT = Npad // NS
  NRC = RPT // RB

  mesh = plsc.VectorSubcoreMesh(
      core_axis_name="c", subcore_axis_name="s", num_cores=NC, num_subcores=NS
  )

  @functools.partial(
      pl.kernel,
      out_type=(
          jax.ShapeDtypeStruct((NC, Npad, DH), F32),
          jax.ShapeDtypeStruct((NC, Npad, DH), F32),
          jax.ShapeDtypeStruct((NC, Npad, DH), F32),
      ),
      mesh=mesh,
      compiler_params=pltpu.CompilerParams(needs_layout_passes=False, use_tc_tiling_on_sc=False),
      scratch_types=dict(
          acc=pltpu.VMEM_SHARED((Npad, DH), F32),
          srcb=pltpu.VMEM((SB, RB), I32),
          dstb=pltpu.VMEM((SB, RB), I32),
          wb=pltpu.VMEM((SB * RB,), F32),
          rows=pltpu.VMEM((RB, DH), F32),
          accb=pltpu.VMEM((RB, DH), F32),
          sumb=pltpu.VMEM((RB, DH), F32),
          zb=pltpu.VMEM((RB, DH), F32),
      ),
  )
  def k(x0, src3, dst3, w3, xsum, xa, xb, acc, srcb, dstb, wb, rows, accb,
        sumb, zb):
    c = lax.axis_index("c")
    s = lax.axis_index("s")
    row0_t = s * RPT

    zeros = jnp.zeros((LANES,), F32)

    def zb_body(r, carry):
      zb[r, pl.ds(0, LANES)] = zeros
      zb[r, pl.ds(LANES, LANES)] = zeros
      return carry

    lax.fori_loop(0, RB, zb_body, 0)

    def zacc_body(kk, carry):
      pltpu.sync_copy(zb, acc.at[pl.ds(row0_t + kk * RB, RB), :])
      return carry

    lax.fori_loop(0, NRC, zacc_body, 0)
    plsc.subcore_barrier()

    splat_dnums = lax.GatherDimensionNumbers(
        offset_dims=(), collapsed_slice_dims=(0,), start_index_map=(0,)
    )
    splat_idx = [
        jnp.full((LANES, 1), j, I32) for j in range(LANES)
    ]

    def splat(vec, j):
      return lax.gather(
          vec,
          splat_idx[j],
          splat_dnums,
          (1,),
          mode=lax.GatherScatterMode.PROMISE_IN_BOUNDS,
      )

    def edge_pass(xprev):
      def sb_body(q, carry):
        base = q * SB
        pltpu.sync_copy(src3.at[s, pl.ds(base, SB), :], srcb)
        pltpu.sync_copy(dst3.at[s, pl.ds(base, SB), :], dstb)
        pltpu.sync_copy(w3.at[s, pl.ds(base * RB, SB * RB)], wb)

        for b in range(SB):
          pltpu.sync_copy(xprev.at[c].at[srcb.at[b]], rows)

          def g_body(g, carry2, b=b):
            e0 = g * LANES
            wvec = wb[pl.ds(b * RB + e0, LANES)]
            for j in range(LANES):
              w = splat(wvec, j)
              r0 = rows[e0 + j, pl.ds(0, LANES)]
              rows[e0 + j, pl.ds(0, LANES)] = r0 * w
              r1 = rows[e0 + j, pl.ds(LANES, LANES)]
              rows[e0 + j, pl.ds(LANES, LANES)] = r1 * w
            return carry2

          lax.fori_loop(0, RB // LANES, g_body, 0)
          pltpu.sync_copy(rows, acc.at[dstb.at[b]], add=True)
        return carry

      lax.fori_loop(0, NSB, sb_body, 0)

    def writeback(layer, xnew):
      final = layer == N_LAYERS - 1
      scale = jnp.float32(1.0 / (N_LAYERS + 1))

      def k_body(kk, carry):
        row0 = row0_t + kk * RB
        pltpu.sync_copy(acc.at[pl.ds(row0, RB), :], accb)
        pltpu.sync_copy(zb, acc.at[pl.ds(row0, RB), :])
        if not final:
          pltpu.sync_copy(accb, xnew.at[c, pl.ds(row0, RB), :])
        if layer == 0:
          pltpu.sync_copy(x0.at[c, pl.ds(row0, RB), :], sumb)
        else:
          pltpu.sync_copy(xsum.at[c, pl.ds(row0, RB), :], sumb)

        def r_body(r, carry2):
          t0 = sumb[r, pl.ds(0, LANES)] + accb[r, pl.ds(0, LANES)]
          t1 = sumb[r, pl.ds(LANES, LANES)] + accb[r, pl.ds(LANES, LANES)]
          if final:
            t0 = t0 * scale
            t1 = t1 * scale
          sumb[r, pl.ds(0, LANES)] = t0
          sumb[r, pl.ds(LANES, LANES)] = t1
          return carry2

        lax.fori_loop(0, RB, r_body, 0)
        pltpu.sync_copy(sumb, xsum.at[c, pl.ds(row0, RB), :])
        return carry

      lax.fori_loop(0, NRC, k_body, 0)

    xprevs = [x0, xa, xb]
    xnews = [xa, xb, xa]
    for layer in range(N_LAYERS):
      edge_pass(xprevs[layer])
      plsc.subcore_barrier()
      writeback(layer, xnews[layer])
      plsc.subcore_barrier()

  return k


@jax.jit
def _lightgcn(user_emb, item_emb, edge_index, edge_weight):
  nu = user_emb.shape[0]
  ni = item_emb.shape[0]
  n = nu + ni
  ego = jnp.concatenate([user_emb, item_emb], axis=0)
  npad = -(-n // (NS * RB)) * (NS * RB)
  ego = jnp.pad(ego, ((0, npad - n), (0, 0)))
  x0 = jnp.stack([ego[:, :DH], ego[:, DH:]], axis=0)

  e = edge_index.shape[1]
  nbp = -(-e // (NS * RB * SB)) * SB
  epad = nbp * NS * RB
  src = jnp.pad(edge_index[0], (0, epad - e)).reshape(NS, nbp, RB)
  dst = jnp.pad(edge_index[1], (0, epad - e)).reshape(NS, nbp, RB)
  w = jnp.pad(edge_weight, (0, epad - e)).reshape(NS, nbp * RB)

  xsum, _, _ = _build(npad, nbp)(x0, src, dst, w)
  mean = jnp.concatenate([xsum[0, :n], xsum[1, :n]], axis=1)
  return mean[:nu], mean[nu:]


def kernel(user_emb, item_emb, edge_index, edge_weight):
  return _lightgcn(user_emb, item_emb, edge_index, edge_weight)


# speedup vs baseline: 6.8394x; 1.2244x over previous
import functools

import jax
import jax.numpy as jnp
from jax import lax
from jax.experimental import pallas as pl
from jax.experimental.pallas import tpu as pltpu
from jax.experimental.pallas import tpu_sc as plsc

NC = 2
NS = 16
LANES = 16
DH = 32
RB = 128
SB = 16
NBUF = 4
N_LAYERS = 3
F32 = jnp.float32
I32 = jnp.int32


@functools.lru_cache(maxsize=None)
def _build(Npad, NBp):
  NSB = NBp // SB
  RPT = Npad // NS
  NRC = RPT // RB

  mesh = plsc.VectorSubcoreMesh(
      core_axis_name="c", subcore_axis_name="s", num_cores=NC, num_subcores=NS
  )

  @functools.partial(
      pl.kernel,
      out_type=(
          jax.ShapeDtypeStruct((NC, Npad, DH), F32),
          jax.ShapeDtypeStruct((NC, Npad, DH), F32),
          jax.ShapeDtypeStruct((NC, Npad, DH), F32),
      ),
      mesh=mesh,
      compiler_params=pltpu.CompilerParams(needs_layout_passes=False, use_tc_tiling_on_sc=False),
      scratch_types=dict(
          acc=pltpu.VMEM_SHARED((Npad, DH), F32),
          srcb=pltpu.VMEM((SB, RB), I32),
          dstb=pltpu.VMEM((SB, RB), I32),
          wb=pltpu.VMEM((SB * RB,), F32),
          rows0=pltpu.VMEM((RB, DH), F32),
          rows1=pltpu.VMEM((RB, DH), F32),
          rows2=pltpu.VMEM((RB, DH), F32),
          rows3=pltpu.VMEM((RB, DH), F32),
          gs0=pltpu.SemaphoreType.DMA,
          gs1=pltpu.SemaphoreType.DMA,
          gs2=pltpu.SemaphoreType.DMA,
          gs3=pltpu.SemaphoreType.DMA,
          ss0=pltpu.SemaphoreType.DMA,
          ss1=pltpu.SemaphoreType.DMA,
          ss2=pltpu.SemaphoreType.DMA,
          ss3=pltpu.SemaphoreType.DMA,
      ),
  )
  def k(x0, src3, dst3, w3, xsum, xa, xb, acc, srcb, dstb, wb, rows0, rows1,
        rows2, rows3, gs0, gs1, gs2, gs3, ss0, ss1, ss2, ss3):
    zb = rows0
    accb = rows1
    sumb = rows2
    c = lax.axis_index("c")
    s = lax.axis_index("s")
    row0_t = s * RPT

    zeros = jnp.zeros((LANES,), F32)

    def zb_body(r, carry):
      zb[r, pl.ds(0, LANES)] = zeros
      zb[r, pl.ds(LANES, LANES)] = zeros
      return carry

    lax.fori_loop(0, RB, zb_body, 0)

    def zacc_body(kk, carry):
      pltpu.sync_copy(zb, acc.at[pl.ds(row0_t + kk * RB, RB), :])
      return carry

    lax.fori_loop(0, NRC, zacc_body, 0)
    plsc.subcore_barrier()

    splat_dnums = lax.GatherDimensionNumbers(
        offset_dims=(), collapsed_slice_dims=(0,), start_index_map=(0,)
    )
    splat_idx = [
        jnp.full((LANES, 1), j, I32) for j in range(LANES)
    ]

    def splat(vec, j):
      return lax.gather(
          vec,
          splat_idx[j],
          splat_dnums,
          (1,),
          mode=lax.GatherScatterMode.PROMISE_IN_BOUNDS,
      )

    def edge_pass(xprev):
      bufs = [rows0, rows1, rows2, rows3]
      gsems = [gs0, gs1, gs2, gs3]
      ssems = [ss0, ss1, ss2, ss3]

      def sb_body(q, carry):
        base = q * SB
        pltpu.sync_copy(src3.at[s, pl.ds(base, SB), :], srcb)
        pltpu.sync_copy(dst3.at[s, pl.ds(base, SB), :], dstb)
        pltpu.sync_copy(w3.at[s, pl.ds(base * RB, SB * RB)], wb)

        gd = {}
        sd = {}

        def start_gather(b):
          cur = b % NBUF
          gd[b] = pltpu.async_copy(
              xprev.at[c].at[srcb.at[b]], bufs[cur], gsems[cur]
          )

        for i in range(NBUF - 1):
          start_gather(i)

        for b in range(SB):
          cur = b % NBUF
          gd[b].wait()
          nb = b + NBUF - 1
          if nb < SB:
            if b >= 1:
              sd[b - 1].wait()
            start_gather(nb)

          def g_body(g, carry2, b=b, rbuf=bufs[cur]):
            e0 = g * LANES
            wvec = wb[pl.ds(b * RB + e0, LANES)]
            for j in range(LANES):
              w = splat(wvec, j)
              r0 = rbuf[e0 + j, pl.ds(0, LANES)]
              rbuf[e0 + j, pl.ds(0, LANES)] = r0 * w
              r1 = rbuf[e0 + j, pl.ds(LANES, LANES)]
              rbuf[e0 + j, pl.ds(LANES, LANES)] = r1 * w
            return carry2

          lax.fori_loop(0, RB // LANES, g_body, 0)
          sd[b] = pltpu.async_copy(
              bufs[cur], acc.at[dstb.at[b]], ssems[cur], add=True
          )
        for b in range(SB - NBUF + 1, SB):
          sd[b].wait()
        return carry

      lax.fori_loop(0, NSB, sb_body, 0)

    def writeback(layer, xnew):
      final = layer == N_LAYERS - 1
      scale = jnp.float32(1.0 / (N_LAYERS + 1))
      lax.fori_loop(0, RB, zb_body, 0)

      def k_body(kk, carry):
        row0 = row0_t + kk * RB
        pltpu.sync_copy(acc.at[pl.ds(row0, RB), :], accb)
        pltpu.sync_copy(zb, acc.at[pl.ds(row0, RB), :])
        if not final:
          pltpu.sync_copy(accb, xnew.at[c, pl.ds(row0, RB), :])
        if layer == 0:
          pltpu.sync_copy(x0.at[c, pl.ds(row0, RB), :], sumb)
        else:
          pltpu.sync_copy(xsum.at[c, pl.ds(row0, RB), :], sumb)

        def r_body(r, carry2):
          t0 = sumb[r, pl.ds(0, LANES)] + accb[r, pl.ds(0, LANES)]
          t1 = sumb[r, pl.ds(LANES, LANES)] + accb[r, pl.ds(LANES, LANES)]
          if final:
            t0 = t0 * scale
            t1 = t1 * scale
          sumb[r, pl.ds(0, LANES)] = t0
          sumb[r, pl.ds(LANES, LANES)] = t1
          return carry2

        lax.fori_loop(0, RB, r_body, 0)
        pltpu.sync_copy(sumb, xsum.at[c, pl.ds(row0, RB), :])
        return carry

      lax.fori_loop(0, NRC, k_body, 0)

    xprevs = [x0, xa, xb]
    xnews = [xa, xb, xa]
    for layer in range(N_LAYERS):
      edge_pass(xprevs[layer])
      plsc.subcore_barrier()
      writeback(layer, xnews[layer])
      plsc.subcore_barrier()

  return k


@jax.jit
def _lightgcn(user_emb, item_emb, edge_index, edge_weight):
  nu = user_emb.shape[0]
  ni = item_emb.shape[0]
  n = nu + ni
  ego = jnp.concatenate([user_emb, item_emb], axis=0)
  npad = -(-n // (NS * RB)) * (NS * RB)
  ego = jnp.pad(ego, ((0, npad - n), (0, 0)))
  x0 = jnp.stack([ego[:, :DH], ego[:, DH:]], axis=0)

  e = edge_index.shape[1]
  nbp = -(-e // (NS * RB * SB)) * SB
  epad = nbp * NS * RB
  src = jnp.pad(edge_index[0], (0, epad - e)).reshape(NS, nbp, RB)
  dst = jnp.pad(edge_index[1], (0, epad - e)).reshape(NS, nbp, RB)
  w = jnp.pad(edge_weight, (0, epad - e)).reshape(NS, nbp * RB)

  xsum, _, _ = _build(npad, nbp)(x0, src, dst, w)
  mean = jnp.concatenate([xsum[0, :n], xsum[1, :n]], axis=1)
  return mean[:nu], mean[nu:]


def kernel(user_emb, item_emb, edge_index, edge_weight):
  return _lightgcn(user_emb, item_emb, edge_index, edge_weight)
